# baseline (speedup 1.0000x reference)
import functools

import jax
import jax.numpy as jnp
from jax import lax
from jax.experimental import pallas as pl
from jax.experimental.pallas import tpu as pltpu
from jax.experimental.pallas import tpu_sc as plsc

M = 1_000_000
D = 64
B = 16_384
NC = 2
NS = 16
NW = NC * NS
BW = B // NW
JC = B // NS
K = 24
TH = 500_224
TC_C = 512


def _body(mem_hbm, val_hbm, idx_hbm, r_hbm, out_hbm,
          stamp, r2, ng2, sidx2, sj2, rows, sem):
    cid = lax.axis_index("c")
    sid = lax.axis_index("s")
    wid = sid * NC + cid
    base = wid * BW
    iota16 = lax.iota(jnp.int32, 16)

    for k in range(4):
        pltpu.sync_copy(r_hbm.at[pl.ds(base + k * 128, 128)], r2.at[k])

    for a in range(4):
        for b in range(8):
            ng2[a, pl.ds(b * 16, 16)] = jnp.full((16,), -1, jnp.int32)
    for k in range(4):
        pltpu.sync_copy(ng2.at[k], stamp.at[r2.at[k]])

    jbase = sid * JC
    for k in range(8):
        pltpu.sync_copy(idx_hbm.at[pl.ds(jbase + k * 128, 128)], sidx2.at[k])
    for a in range(8):
        for b in range(8):
            sj2[a, pl.ds(b * 16, 16)] = jbase + a * 128 + b * 16 + iota16

    plsc.subcore_barrier()

    for t in range(NS):
        @pl.when(sid == t)
        def _scatter(_t=t):
            for k in range(8):
                pltpu.sync_copy(sj2.at[k], stamp.at[sidx2.at[k]])
        plsc.subcore_barrier()

    for k in range(4):
        pltpu.sync_copy(stamp.at[r2.at[k]], ng2.at[k])

    HB = BW // 2
    NG = HB // 16
    for p in range(2):
        def _grp(q, _, _p=p):
            @pl.when(q < NG)
            def _start():
                fb = _p * HB + q * 16
                a, off = fb >> 7, fb & 127
                vr = r2[a, pl.ds(off, 16)]
                vg = ng2[a, pl.ds(off, 16)]
                vpr = jnp.where(vr >= TH, vr - TH, vr)
                for u in range(16):
                    g, r = vg[u], vpr[u]

                    @pl.when(g >= 0)
                    def _from_val(_g=g, _u=u):
                        pltpu.async_copy(val_hbm.at[_g >> 1],
                                         rows.at[q * 16 + _u], sem)

                    @pl.when(g < 0)
                    def _from_mem(_r=r, _u=u):
                        pltpu.async_copy(mem_hbm.at[_r],
                                         rows.at[q * 16 + _u], sem)

            @pl.when(q >= 1)
            def _drain():
                pltpu.make_async_copy(
                    mem_hbm.at[pl.ds(0, 16)],
                    rows.at[pl.ds((q - 1) * 16, 16)], sem).wait()

            return _

        lax.fori_loop(0, NG + 1, _grp, None)

        def _pack(q, _, _p=p):
            fb = _p * HB + q * 16
            a = fb >> 7
            off = fb & 127
            vr = r2[a, pl.ds(off, 16)]
            vg = ng2[a, pl.ds(off, 16)]
            hv = jnp.where(vg >= 0, vg & 1,
                           jnp.where(vr >= TH, 1, 0).astype(jnp.int32))
            for u in range(16):
                f = q * 16 + u
                dst, dl = q * 8 + (u >> 1), (u & 1) * 64

                @pl.when(hv[u] == 1)
                def _hi(_f=f, _dst=dst, _dl=dl):
                    for cb in range(4):
                        rows[_dst, pl.ds(_dl + cb * 16, 16)] = (
                            rows[_f, pl.ds(64 + cb * 16, 16)])

                @pl.when(hv[u] == 0)
                def _lo(_f=f, _dst=dst, _dl=dl):
                    for cb in range(4):
                        rows[_dst, pl.ds(_dl + cb * 16, 16)] = (
                            rows[_f, pl.ds(cb * 16, 16)])
            return _

        lax.fori_loop(0, NG, _pack, None)

        obase = pl.multiple_of((base >> 1) + p * (HB // 2), 8)
        pltpu.sync_copy(rows.at[pl.ds(0, HB // 2)],
                        out_hbm.at[pl.ds(obase, HB // 2)])


_sc_call = functools.partial(
    pl.kernel,
    out_type=jax.ShapeDtypeStruct((B // 2, 2 * D), jnp.float32),
    mesh=plsc.VectorSubcoreMesh(core_axis_name="c", subcore_axis_name="s",
                                num_cores=NC, num_subcores=NS),
    scratch_types=[
        pltpu.VMEM_SHARED((1_000_000,), jnp.int32),
        pltpu.VMEM((4, 128), jnp.int32),
        pltpu.VMEM((4, 128), jnp.int32),
        pltpu.VMEM((8, 128), jnp.int32),
        pltpu.VMEM((8, 128), jnp.int32),
        pltpu.VMEM((BW // 2, 128), jnp.float32),
        pltpu.SemaphoreType.DMA,
    ],
)(_body)


def _t_body(a_ref, b_ref, o_ref):
    o_ref[:, 0:D] = a_ref[...].T
    o_ref[:, D:2 * D] = b_ref[...].T


_t_grid = TH // TC_C
_table = pl.pallas_call(
    _t_body,
    grid=(_t_grid,),
    in_specs=[pl.BlockSpec((D, TC_C), lambda i: (0, i)),
              pl.BlockSpec((D, TC_C), lambda i: (0, i + _t_grid))],
    out_specs=pl.BlockSpec((TC_C, 2 * D), lambda i: (i, 0)),
    out_shape=jax.ShapeDtypeStruct((TH, 2 * D), jnp.float32),
)


def kernel(mem, val, idx, retrieve_idx):
    mem_t = jnp.swapaxes(mem, 0, 1)
    table = _table(mem_t, mem_t)
    packed = _sc_call(table,
                      jnp.reshape(val, (B // 2, 2 * D)),
                      idx.astype(jnp.int32), retrieve_idx.astype(jnp.int32))
    return jnp.reshape(packed, (B, D))


# speedup vs baseline: 2.0182x; 2.0182x over previous
import functools

import jax
import jax.numpy as jnp
from jax import lax
from jax.experimental import pallas as pl
from jax.experimental.pallas import tpu as pltpu
from jax.experimental.pallas import tpu_sc as plsc

M = 1_000_000
D = 64
B = 16_384
NC = 2
NS = 16
NW = NC * NS
BW = B // NW
JC = B // NS


def _jbody(idx_hbm, r_hbm, g_hbm, stamp, r2, ng2, sidx2, sj2):
    cid = lax.axis_index("c")
    sid = lax.axis_index("s")
    wid = sid * NC + cid
    base = wid * BW
    iota16 = lax.iota(jnp.int32, 16)

    for k in range(4):
        pltpu.sync_copy(r_hbm.at[pl.ds(base + k * 128, 128)], r2.at[k])

    for a in range(4):
        for b in range(8):
            ng2[a, pl.ds(b * 16, 16)] = jnp.full((16,), -1, jnp.int32)
    for k in range(4):
        pltpu.sync_copy(ng2.at[k], stamp.at[r2.at[k]])

    jbase = sid * JC
    for k in range(8):
        pltpu.sync_copy(idx_hbm.at[pl.ds(jbase + k * 128, 128)], sidx2.at[k])
    for a in range(8):
        for b in range(8):
            sj2[a, pl.ds(b * 16, 16)] = jbase + a * 128 + b * 16 + iota16

    plsc.subcore_barrier()

    for t in range(NS):
        @pl.when(sid == t)
        def _scatter(_t=t):
            for k in range(8):
                pltpu.sync_copy(sj2.at[k], stamp.at[sidx2.at[k]])
        plsc.subcore_barrier()

    for k in range(4):
        pltpu.sync_copy(stamp.at[r2.at[k]], ng2.at[k])
    for k in range(4):
        pltpu.sync_copy(ng2.at[k], g_hbm.at[pl.ds(base + k * 128, 128)])


def _fbody(mem_hbm, val_hbm, g_hbm, r_hbm, out_hbm, r2, ng2, rows, sem):
    cid = lax.axis_index("c")
    sid = lax.axis_index("s")
    wid = sid * NC + cid
    base = wid * BW

    for k in range(4):
        pltpu.sync_copy(r_hbm.at[pl.ds(base + k * 128, 128)], r2.at[k])
    for k in range(4):
        pltpu.sync_copy(g_hbm.at[pl.ds(base + k * 128, 128)], ng2.at[k])

    HB = BW // 2
    NG = HB // 16
    for p in range(2):
        def _grp(q, _, _p=p):
            @pl.when(q < NG)
            def _start():
                fb = _p * HB + q * 16
                a, off = fb >> 7, fb & 127
                vr = r2[a, pl.ds(off, 16)]
                vg = ng2[a, pl.ds(off, 16)]
                for u in range(16):
                    g, r = vg[u], vr[u]

                    @pl.when(g >= 0)
                    def _from_val(_g=g, _u=u):
                        pltpu.async_copy(val_hbm.at[_g],
                                         rows.at[q * 16 + _u], sem)

                    @pl.when(g < 0)
                    def _from_mem(_r=r, _u=u):
                        pltpu.async_copy(mem_hbm.at[_r],
                                         rows.at[q * 16 + _u], sem)

            @pl.when(q >= 1)
            def _drain():
                pltpu.make_async_copy(
                    mem_hbm.at[pl.ds(0, 16)],
                    rows.at[pl.ds((q - 1) * 16, 16)], sem).wait()

            return _

        lax.fori_loop(0, NG + 1, _grp, None)
        pltpu.sync_copy(rows, out_hbm.at[pl.ds(base + p * HB, HB)])


_mesh = plsc.VectorSubcoreMesh(core_axis_name="c", subcore_axis_name="s",
                               num_cores=NC, num_subcores=NS)

_join_call = functools.partial(
    pl.kernel,
    out_type=jax.ShapeDtypeStruct((B,), jnp.int32),
    mesh=_mesh,
    scratch_types=[
        pltpu.VMEM_SHARED((1_000_000,), jnp.int32),
        pltpu.VMEM((4, 128), jnp.int32),
        pltpu.VMEM((4, 128), jnp.int32),
        pltpu.VMEM((8, 128), jnp.int32),
        pltpu.VMEM((8, 128), jnp.int32),
    ],
)(_jbody)

_fetch_call = functools.partial(
    pl.kernel,
    out_type=jax.ShapeDtypeStruct((B, D), jnp.float32),
    mesh=_mesh,
    scratch_types=[
        pltpu.VMEM((4, 128), jnp.int32),
        pltpu.VMEM((4, 128), jnp.int32),
        pltpu.VMEM((BW // 2, D), jnp.float32),
        pltpu.SemaphoreType.DMA,
    ],
)(_fbody)


def kernel(mem, val, idx, retrieve_idx):
    idx32 = idx.astype(jnp.int32)
    r32 = retrieve_idx.astype(jnp.int32)
    g = _join_call(idx32, r32)
    return _fetch_call(mem, val, g, r32)
